# Initial kernel scaffold; baseline (speedup 1.0000x reference)
import jax
import jax.numpy as jnp
from jax.experimental import pallas as pl


def kernel(scores, emb_weight):
    raise NotImplementedError("write your pallas kernel here")


# speedup vs baseline: 1.7302x; 1.7302x over previous
import functools

import jax
import jax.numpy as jnp
from jax import lax
from jax.experimental import pallas as pl
from jax.experimental.pallas import tpu as pltpu
from jax.experimental.pallas import tpu_sc as plsc

N = 1024
VOCAB = 100000
DIM = 64

BN = 256
BV = 2048
NBV = (VOCAB + BV - 1) // BV

INT_MAX = 2**31 - 1


def _argmax_body(s_ref, best_ref, mval_ref, marg_ref):
    j = pl.program_id(1)
    vals = s_ref[...]
    col = lax.broadcasted_iota(jnp.int32, (BN, BV), 1) + j * BV
    valid = col < VOCAB
    v = jnp.where(valid, vals, -jnp.inf)
    m = jnp.max(v, axis=1, keepdims=True)
    cand = jnp.where(v == m, col, INT_MAX)
    a = jnp.min(cand, axis=1, keepdims=True)

    @pl.when(j == 0)
    def _():
        mval_ref[...] = m
        marg_ref[...] = a

    @pl.when(j > 0)
    def _():
        better = m > mval_ref[...]
        mval_ref[...] = jnp.where(better, m, mval_ref[...])
        marg_ref[...] = jnp.where(better, a, marg_ref[...])

    @pl.when(j == NBV - 1)
    def _():
        best_ref[...] = marg_ref[...]


_argmax_call = pl.pallas_call(
    _argmax_body,
    grid=(N // BN, NBV),
    in_specs=[pl.BlockSpec((BN, BV), lambda i, j: (i, j))],
    out_specs=pl.BlockSpec((BN, 1), lambda i, j: (i, 0)),
    out_shape=jax.ShapeDtypeStruct((N, 1), jnp.int32),
    scratch_shapes=[
        pltpu.VMEM((BN, 1), jnp.float32),
        pltpu.VMEM((BN, 1), jnp.int32),
    ],
    compiler_params=pltpu.CompilerParams(
        dimension_semantics=("parallel", "arbitrary"),
    ),
)


NC, NS = 2, 16
NW = NC * NS
BPW = N // NW

@functools.cache
def _make_gather_sc():
    mesh = plsc.VectorSubcoreMesh(core_axis_name="c", subcore_axis_name="s")

    @functools.partial(
        pl.kernel,
        mesh=mesh,
        out_type=jax.ShapeDtypeStruct((N, DIM), jnp.float32),
        scratch_types=[
            pltpu.VMEM((BPW,), jnp.int32),
            pltpu.VMEM((BPW, DIM), jnp.float32),
            pltpu.SemaphoreType.DMA,
        ],
        compiler_params=pltpu.CompilerParams(use_tc_tiling_on_sc=False),
    )
    def _gather_sc(table_hbm, idx_hbm, out_hbm, idx_v, rows_v, sem):
        wid = lax.axis_index("s") * NC + lax.axis_index("c")
        base = wid * BPW
        pltpu.sync_copy(idx_hbm.at[pl.ds(base, BPW)], idx_v)
        pltpu.async_copy(table_hbm.at[idx_v], rows_v, sem).wait()
        pltpu.sync_copy(rows_v, out_hbm.at[pl.ds(base, BPW)])

    return _gather_sc


def kernel(scores, emb_weight):
    best = _argmax_call(scores).reshape(N)
    emb = _make_gather_sc()(emb_weight, best)
    return emb, best


# speedup vs baseline: 1.9914x; 1.1510x over previous
import functools

import jax
import jax.numpy as jnp
from jax import lax
from jax.experimental import pallas as pl
from jax.experimental.pallas import tpu as pltpu
from jax.experimental.pallas import tpu_sc as plsc

N = 1024
VOCAB = 100000
DIM = 64

BN = 512
BV = 4096
NBV = (VOCAB + BV - 1) // BV

INT_MAX = 2**31 - 1


def _argmax_body(s_ref, best_ref, mval_ref, marg_ref):
    j = pl.program_id(1)
    vals = s_ref[...]
    col = lax.broadcasted_iota(jnp.int32, (BN, BV), 1) + j * BV
    valid = col < VOCAB
    v = jnp.where(valid, vals, -jnp.inf)
    m = jnp.max(v, axis=1, keepdims=True)
    cand = jnp.where(v == m, col, INT_MAX)
    a = jnp.min(cand, axis=1, keepdims=True)

    @pl.when(j == 0)
    def _():
        mval_ref[...] = m
        marg_ref[...] = a

    @pl.when(j > 0)
    def _():
        better = m > mval_ref[...]
        mval_ref[...] = jnp.where(better, m, mval_ref[...])
        marg_ref[...] = jnp.where(better, a, marg_ref[...])

    @pl.when(j == NBV - 1)
    def _():
        best_ref[...] = marg_ref[...]


_argmax_call = pl.pallas_call(
    _argmax_body,
    grid=(N // BN, NBV),
    in_specs=[pl.BlockSpec((BN, BV), lambda i, j: (i, j))],
    out_specs=pl.BlockSpec((BN, 1), lambda i, j: (i, 0)),
    out_shape=jax.ShapeDtypeStruct((N, 1), jnp.int32),
    scratch_shapes=[
        pltpu.VMEM((BN, 1), jnp.float32),
        pltpu.VMEM((BN, 1), jnp.int32),
    ],
    compiler_params=pltpu.CompilerParams(
        dimension_semantics=("parallel", "arbitrary"),
    ),
)


NC, NS = 2, 16
NW = NC * NS
BPW = N // NW

@functools.cache
def _make_gather_sc():
    mesh = plsc.VectorSubcoreMesh(core_axis_name="c", subcore_axis_name="s")

    @functools.partial(
        pl.kernel,
        mesh=mesh,
        out_type=jax.ShapeDtypeStruct((N, DIM), jnp.float32),
        scratch_types=[
            pltpu.VMEM((BPW,), jnp.int32),
            pltpu.VMEM((BPW, DIM), jnp.float32),
            pltpu.SemaphoreType.DMA,
        ],
        compiler_params=pltpu.CompilerParams(use_tc_tiling_on_sc=False),
    )
    def _gather_sc(table_hbm, idx_hbm, out_hbm, idx_v, rows_v, sem):
        wid = lax.axis_index("s") * NC + lax.axis_index("c")
        base = wid * BPW
        pltpu.sync_copy(idx_hbm.at[pl.ds(base, BPW)], idx_v)
        pltpu.async_copy(table_hbm.at[idx_v], rows_v, sem).wait()
        pltpu.sync_copy(rows_v, out_hbm.at[pl.ds(base, BPW)])

    return _gather_sc


def kernel(scores, emb_weight):
    best = _argmax_call(scores).reshape(N)
    emb = _make_gather_sc()(emb_weight, best)
    return emb, best


# speedup vs baseline: 2.0322x; 1.0205x over previous
import functools

import jax
import jax.numpy as jnp
from jax import lax
from jax.experimental import pallas as pl
from jax.experimental.pallas import tpu as pltpu
from jax.experimental.pallas import tpu_sc as plsc

N = 1024
VOCAB = 100000
DIM = 64

BN = 1024
BV = 4096
NBV = (VOCAB + BV - 1) // BV

INT_MAX = 2**31 - 1


def _argmax_body(s_ref, best_ref, mval_ref, marg_ref):
    j = pl.program_id(1)
    vals = s_ref[...]
    col = lax.broadcasted_iota(jnp.int32, (BN, BV), 1) + j * BV
    valid = col < VOCAB
    v = jnp.where(valid, vals, -jnp.inf)
    m = jnp.max(v, axis=1, keepdims=True)
    cand = jnp.where(v == m, col, INT_MAX)
    a = jnp.min(cand, axis=1, keepdims=True)

    @pl.when(j == 0)
    def _():
        mval_ref[...] = m
        marg_ref[...] = a

    @pl.when(j > 0)
    def _():
        better = m > mval_ref[...]
        mval_ref[...] = jnp.where(better, m, mval_ref[...])
        marg_ref[...] = jnp.where(better, a, marg_ref[...])

    @pl.when(j == NBV - 1)
    def _():
        best_ref[...] = marg_ref[...]


_argmax_call = pl.pallas_call(
    _argmax_body,
    grid=(N // BN, NBV),
    in_specs=[pl.BlockSpec((BN, BV), lambda i, j: (i, j))],
    out_specs=pl.BlockSpec((BN, 1), lambda i, j: (i, 0)),
    out_shape=jax.ShapeDtypeStruct((N, 1), jnp.int32),
    scratch_shapes=[
        pltpu.VMEM((BN, 1), jnp.float32),
        pltpu.VMEM((BN, 1), jnp.int32),
    ],
    compiler_params=pltpu.CompilerParams(
        dimension_semantics=("parallel", "arbitrary"),
    ),
)


NC, NS = 2, 16
NW = NC * NS
BPW = N // NW

@functools.cache
def _make_gather_sc():
    mesh = plsc.VectorSubcoreMesh(core_axis_name="c", subcore_axis_name="s")

    @functools.partial(
        pl.kernel,
        mesh=mesh,
        out_type=jax.ShapeDtypeStruct((N, DIM), jnp.float32),
        scratch_types=[
            pltpu.VMEM((BPW,), jnp.int32),
            pltpu.VMEM((BPW, DIM), jnp.float32),
            pltpu.SemaphoreType.DMA,
        ],
        compiler_params=pltpu.CompilerParams(use_tc_tiling_on_sc=False),
    )
    def _gather_sc(table_hbm, idx_hbm, out_hbm, idx_v, rows_v, sem):
        wid = lax.axis_index("s") * NC + lax.axis_index("c")
        base = wid * BPW
        pltpu.sync_copy(idx_hbm.at[pl.ds(base, BPW)], idx_v)
        pltpu.async_copy(table_hbm.at[idx_v], rows_v, sem).wait()
        pltpu.sync_copy(rows_v, out_hbm.at[pl.ds(base, BPW)])

    return _gather_sc


def kernel(scores, emb_weight):
    best = _argmax_call(scores).reshape(N)
    emb = _make_gather_sc()(emb_weight, best)
    return emb, best
